# baseline (speedup 1.0000x reference)
import jax
import jax.numpy as jnp
import numpy as np
from jax import lax
from jax.experimental import pallas as pl
from jax.experimental.pallas import tpu as pltpu
from jax.experimental.pallas import tpu_sc as plsc

_NC = 2
_NS = 16
_NW = _NC * _NS
_C = 64
_G = 64
_TSLOW = 112
_TFAST = 208
_SLOW_CID = 0


def _edge_mlp(ea, w1, b1, w2, b2):
    E, K = ea.shape
    H = w2.shape[1]
    BE = 4096

    def body(ea_ref, w1_ref, b1_ref, w2_ref, b2_ref, out_ref):
        t = jnp.maximum(ea_ref[...] @ w1_ref[...] + b1_ref[...], 0.0)
        out_ref[...] = jnp.maximum(t @ w2_ref[...] + b2_ref[...], 0.0)

    return pl.pallas_call(
        body,
        grid=(E // BE,),
        in_specs=[
            pl.BlockSpec((BE, K), lambda i: (i, 0)),
            pl.BlockSpec(w1.shape, lambda i: (0, 0)),
            pl.BlockSpec(b1.shape, lambda i: (0, 0)),
            pl.BlockSpec(w2.shape, lambda i: (0, 0)),
            pl.BlockSpec(b2.shape, lambda i: (0, 0)),
        ],
        out_specs=pl.BlockSpec((BE, H), lambda i: (i, 0)),
        out_shape=jax.ShapeDtypeStruct((E, H), jnp.float32),
    )(ea, w1, b1, w2, b2)


def _pack16(a):
    r, d = a.shape
    t = a.reshape(r, d // 32, 2, 16).transpose(0, 1, 3, 2)
    return lax.bitcast_convert_type(t, jnp.int32).reshape(r, d // 2)


def _msg_agg(h, e, idxp, zrows):
    n, hdim = h.shape
    rps = zrows.shape[0]
    npad = rps * _NS
    KU = 4
    mesh = plsc.VectorSubcoreMesh(core_axis_name="c", subcore_axis_name="s")

    def body(h_hbm, e_hbm, idx_hbm, z_hbm, out_hbm,
             idx_v, srcb, dstb, h0, h1, e0, e1,
             agg_sh, hs0, hs1, es0, es1):
        cid = lax.axis_index("c")
        sid = lax.axis_index("s")
        slow = cid == _SLOW_CID
        tw = jnp.where(slow, _TSLOW, _TFAST)
        cstart = pl.multiple_of(jnp.where(slow, sid * _TSLOW,
                                          _NS * _TSLOW + sid * _TFAST), 16)
        estart = pl.multiple_of(cstart * _C, 1024)

        pltpu.sync_copy(z_hbm, agg_sh.at[pl.ds(sid * rps, rps)])
        pltpu.sync_copy(
            idx_hbm.at[pl.ds(pl.multiple_of(cstart // 2, 8), _TFAST // 2)],
            idx_v)
        plsc.subcore_barrier()

        hb = (h0, h1)
        eb = (e0, e1)
        hs = (hs0, hs1)
        es = (es0, es1)

        def unpack(q, b):
            row = q // 2
            for u in range(_C // 16):
                p = idx_v[row, pl.ds(b * _C + u * 16, 16)]
                sl = pl.ds(u * 16, 16)
                srcb[b, sl] = p & 0xFFFF
                dstb[b, sl] = lax.shift_right_logical(p, 16)

        def step(t, b, prefetch):
            pltpu.make_async_copy(
                e_hbm.at[pl.ds(estart + t * _C, _C)], eb[b], es[b]).wait()
            pltpu.make_async_copy(h_hbm.at[srcb.at[b]], hb[b], hs[b]).wait()
            hv, ev = hb[b], eb[b]

            def rowblk(i, c2):
                for u in range(KU):
                    r = i * KU + u
                    for cc in range(hdim // 16):
                        sl = pl.ds(cc * 16, 16)
                        hv[r, sl] = jnp.maximum(hv[r, sl] + ev[r, sl], 0.0)
                return c2

            lax.fori_loop(0, _C // KU, rowblk, 0)
            if prefetch:
                pltpu.async_copy(
                    e_hbm.at[pl.ds(estart + (t + 2) * _C, _C)], ev, es[b])
            pltpu.sync_copy(hv, agg_sh.at[dstb.at[b]], add=True)
            if prefetch:
                unpack(t + 2, b)
                pltpu.async_copy(h_hbm.at[srcb.at[b]], hv, hs[b])

        for b in (0, 1):
            unpack(b, b)
            pltpu.async_copy(
                e_hbm.at[pl.ds(estart + b * _C, _C)], eb[b], es[b])
            pltpu.async_copy(h_hbm.at[srcb.at[b]], hb[b], hs[b])

        def pair(k, c):
            step(2 * k, 0, True)
            step(2 * k + 1, 1, True)
            return c

        lax.fori_loop(0, tw // 2 - 1, pair, 0)
        step(tw - 2, 0, False)
        step(tw - 1, 1, False)

        plsc.subcore_barrier()
        pltpu.sync_copy(agg_sh.at[pl.ds(sid * rps, rps)],
                        out_hbm.at[cid, pl.ds(sid * rps, rps)])

    f = pl.kernel(
        body,
        out_type=jax.ShapeDtypeStruct((_NC, npad, hdim), jnp.float32),
        mesh=mesh,
        scratch_types=[
            pltpu.VMEM((_TFAST // 2, 2 * _C), jnp.int32),
            pltpu.VMEM((2, _C), jnp.int32),
            pltpu.VMEM((2, _C), jnp.int32),
            pltpu.VMEM((_C, hdim), jnp.float32),
            pltpu.VMEM((_C, hdim), jnp.float32),
            pltpu.VMEM((_C, hdim), jnp.float32),
            pltpu.VMEM((_C, hdim), jnp.float32),
            pltpu.VMEM_SHARED((npad, hdim), jnp.float32),
            pltpu.SemaphoreType.DMA,
            pltpu.SemaphoreType.DMA,
            pltpu.SemaphoreType.DMA,
            pltpu.SemaphoreType.DMA,
        ],
    )
    return f(h, e, idxp, zrows)


def _update(h, agg2, eps11, w1, b1, w2, b2):
    n, hdim = h.shape
    BN = 2000

    def body(h_ref, a_ref, eps_ref, w1_ref, b1_ref, w2_ref, b2_ref, out_ref):
        scale = 1.0 + eps_ref[0, 0]
        z = scale * h_ref[...] + a_ref[0] + a_ref[1]
        z = jnp.maximum(z @ w1_ref[...] + b1_ref[...], 0.0)
        out_ref[...] = jnp.maximum(z @ w2_ref[...] + b2_ref[...], 0.0)

    return pl.pallas_call(
        body,
        grid=(n // BN,),
        in_specs=[
            pl.BlockSpec((BN, hdim), lambda i: (i, 0)),
            pl.BlockSpec((2, BN, hdim), lambda i: (0, i, 0)),
            pl.BlockSpec((1, 1), lambda i: (0, 0), memory_space=pltpu.SMEM),
            pl.BlockSpec((hdim, hdim), lambda i: (0, 0)),
            pl.BlockSpec((1, hdim), lambda i: (0, 0)),
            pl.BlockSpec((hdim, hdim), lambda i: (0, 0)),
            pl.BlockSpec((1, hdim), lambda i: (0, 0)),
        ],
        out_specs=pl.BlockSpec((BN, hdim), lambda i: (i, 0)),
        out_shape=jax.ShapeDtypeStruct((n, hdim), jnp.float32),
    )(h, agg2, eps11, w1, b1, w2, b2)


def _pool_proj(h, batch2, pw, pb):
    n, hdim = h.shape

    def body(h_ref, b_ref, pw_ref, pb_ref, out_ref):
        seg = b_ref[...]
        ids = lax.broadcasted_iota(jnp.int32, (_G, n), 0)
        onehot = jnp.where(seg == ids, 1.0, 0.0)
        hg = jax.lax.dot(onehot, h_ref[...], preferred_element_type=jnp.float32)
        out_ref[...] = jnp.maximum(hg @ pw_ref[...] + pb_ref[...], 0.0)

    return pl.pallas_call(
        body,
        out_shape=jax.ShapeDtypeStruct((_G, hdim), jnp.float32),
    )(h, batch2, pw, pb)


def kernel(x, edge_attr, params, edge_index, batch):
    n, hdim = x.shape
    E = edge_attr.shape[0]
    npad = ((n + 8 * _NS - 1) // (8 * _NS)) * (8 * _NS)
    epad = _NS * (_TSLOW + _TFAST) * _C - E
    ea_p = jnp.concatenate(
        [edge_attr, jnp.zeros((epad, edge_attr.shape[1]), jnp.float32)])
    e = _edge_mlp(ea_p,
                  params["edge_w1"], params["edge_b1"].reshape(1, -1),
                  params["edge_w2"], params["edge_b2"].reshape(1, -1))
    src_p = jnp.concatenate([edge_index[0], jnp.zeros((epad,), jnp.int32)])
    dst_p = jnp.concatenate([edge_index[1],
                             n + (jnp.arange(epad, dtype=jnp.int32) % (npad - n))])
    idxp = (jnp.left_shift(dst_p, 16) | src_p).reshape(-1, 2 * _C)
    zrows = jnp.zeros((npad // _NS, hdim), jnp.float32)
    h = x
    for lp in params["layers"]:
        agg2 = _msg_agg(h, e, idxp, zrows)
        h = _update(h, agg2, lp["eps"].reshape(1, 1),
                    lp["w1"], lp["b1"].reshape(1, -1),
                    lp["w2"], lp["b2"].reshape(1, -1))
    return _pool_proj(h, batch.reshape(1, -1),
                      params["proj_w"], params["proj_b"].reshape(1, -1))


# speedup vs baseline: 1.0460x; 1.0460x over previous
import jax
import jax.numpy as jnp
import numpy as np
from jax import lax
from jax.experimental import pallas as pl
from jax.experimental.pallas import tpu as pltpu
from jax.experimental.pallas import tpu_sc as plsc

_NC = 2
_NS = 16
_NW = _NC * _NS
_C = 64
_G = 64
_TSLOW = 112
_TFAST = 208
_SLOW_CID = 1


def _edge_mlp(ea, w1, b1, w2, b2):
    E, K = ea.shape
    H = w2.shape[1]
    BE = 4096

    def body(ea_ref, w1_ref, b1_ref, w2_ref, b2_ref, out_ref):
        t = jnp.maximum(ea_ref[...] @ w1_ref[...] + b1_ref[...], 0.0)
        out_ref[...] = jnp.maximum(t @ w2_ref[...] + b2_ref[...], 0.0)

    return pl.pallas_call(
        body,
        grid=(E // BE,),
        in_specs=[
            pl.BlockSpec((BE, K), lambda i: (i, 0)),
            pl.BlockSpec(w1.shape, lambda i: (0, 0)),
            pl.BlockSpec(b1.shape, lambda i: (0, 0)),
            pl.BlockSpec(w2.shape, lambda i: (0, 0)),
            pl.BlockSpec(b2.shape, lambda i: (0, 0)),
        ],
        out_specs=pl.BlockSpec((BE, H), lambda i: (i, 0)),
        out_shape=jax.ShapeDtypeStruct((E, H), jnp.float32),
    )(ea, w1, b1, w2, b2)


def _pack16(a):
    r, d = a.shape
    t = a.reshape(r, d // 32, 2, 16).transpose(0, 1, 3, 2)
    return lax.bitcast_convert_type(t, jnp.int32).reshape(r, d // 2)


def _msg_agg(h, e, idxp, zrows):
    n, hdim = h.shape
    rps = zrows.shape[0]
    npad = rps * _NS
    KU = 4
    mesh = plsc.VectorSubcoreMesh(core_axis_name="c", subcore_axis_name="s")

    def body(h_hbm, e_hbm, idx_hbm, z_hbm, out_hbm,
             idx_v, srcb, dstb, h0, h1, e0, e1,
             agg_sh, hs0, hs1, es0, es1):
        cid = lax.axis_index("c")
        sid = lax.axis_index("s")
        slow = cid == _SLOW_CID
        tw = jnp.where(slow, _TSLOW, _TFAST)
        cstart = pl.multiple_of(jnp.where(slow, sid * _TSLOW,
                                          _NS * _TSLOW + sid * _TFAST), 16)
        estart = pl.multiple_of(cstart * _C, 1024)

        pltpu.sync_copy(z_hbm, agg_sh.at[pl.ds(sid * rps, rps)])
        pltpu.sync_copy(
            idx_hbm.at[pl.ds(pl.multiple_of(cstart // 2, 8), _TFAST // 2)],
            idx_v)
        plsc.subcore_barrier()

        hb = (h0, h1)
        eb = (e0, e1)
        hs = (hs0, hs1)
        es = (es0, es1)

        def unpack(q, b):
            row = q // 2
            for u in range(_C // 16):
                p = idx_v[row, pl.ds(b * _C + u * 16, 16)]
                sl = pl.ds(u * 16, 16)
                srcb[b, sl] = p & 0xFFFF
                dstb[b, sl] = lax.shift_right_logical(p, 16)

        def step(t, b, prefetch):
            pltpu.make_async_copy(
                e_hbm.at[pl.ds(estart + t * _C, _C)], eb[b], es[b]).wait()
            pltpu.make_async_copy(h_hbm.at[srcb.at[b]], hb[b], hs[b]).wait()
            hv, ev = hb[b], eb[b]

            def rowblk(i, c2):
                for u in range(KU):
                    r = i * KU + u
                    for cc in range(hdim // 16):
                        sl = pl.ds(cc * 16, 16)
                        hv[r, sl] = jnp.maximum(hv[r, sl] + ev[r, sl], 0.0)
                return c2

            lax.fori_loop(0, _C // KU, rowblk, 0)
            if prefetch:
                pltpu.async_copy(
                    e_hbm.at[pl.ds(estart + (t + 2) * _C, _C)], ev, es[b])
            pltpu.sync_copy(hv, agg_sh.at[dstb.at[b]], add=True)
            if prefetch:
                unpack(t + 2, b)
                pltpu.async_copy(h_hbm.at[srcb.at[b]], hv, hs[b])

        for b in (0, 1):
            unpack(b, b)
            pltpu.async_copy(
                e_hbm.at[pl.ds(estart + b * _C, _C)], eb[b], es[b])
            pltpu.async_copy(h_hbm.at[srcb.at[b]], hb[b], hs[b])

        def pair(k, c):
            step(2 * k, 0, True)
            step(2 * k + 1, 1, True)
            return c

        lax.fori_loop(0, tw // 2 - 1, pair, 0)
        step(tw - 2, 0, False)
        step(tw - 1, 1, False)

        plsc.subcore_barrier()
        pltpu.sync_copy(agg_sh.at[pl.ds(sid * rps, rps)],
                        out_hbm.at[cid, pl.ds(sid * rps, rps)])

    f = pl.kernel(
        body,
        out_type=jax.ShapeDtypeStruct((_NC, npad, hdim), jnp.float32),
        mesh=mesh,
        scratch_types=[
            pltpu.VMEM((_TFAST // 2, 2 * _C), jnp.int32),
            pltpu.VMEM((2, _C), jnp.int32),
            pltpu.VMEM((2, _C), jnp.int32),
            pltpu.VMEM((_C, hdim), jnp.float32),
            pltpu.VMEM((_C, hdim), jnp.float32),
            pltpu.VMEM((_C, hdim), jnp.float32),
            pltpu.VMEM((_C, hdim), jnp.float32),
            pltpu.VMEM_SHARED((npad, hdim), jnp.float32),
            pltpu.SemaphoreType.DMA,
            pltpu.SemaphoreType.DMA,
            pltpu.SemaphoreType.DMA,
            pltpu.SemaphoreType.DMA,
        ],
    )
    return f(h, e, idxp, zrows)


def _update(h, agg2, eps11, w1, b1, w2, b2):
    n, hdim = h.shape
    BN = 2000

    def body(h_ref, a_ref, eps_ref, w1_ref, b1_ref, w2_ref, b2_ref, out_ref):
        scale = 1.0 + eps_ref[0, 0]
        z = scale * h_ref[...] + a_ref[0] + a_ref[1]
        z = jnp.maximum(z @ w1_ref[...] + b1_ref[...], 0.0)
        out_ref[...] = jnp.maximum(z @ w2_ref[...] + b2_ref[...], 0.0)

    return pl.pallas_call(
        body,
        grid=(n // BN,),
        in_specs=[
            pl.BlockSpec((BN, hdim), lambda i: (i, 0)),
            pl.BlockSpec((2, BN, hdim), lambda i: (0, i, 0)),
            pl.BlockSpec((1, 1), lambda i: (0, 0), memory_space=pltpu.SMEM),
            pl.BlockSpec((hdim, hdim), lambda i: (0, 0)),
            pl.BlockSpec((1, hdim), lambda i: (0, 0)),
            pl.BlockSpec((hdim, hdim), lambda i: (0, 0)),
            pl.BlockSpec((1, hdim), lambda i: (0, 0)),
        ],
        out_specs=pl.BlockSpec((BN, hdim), lambda i: (i, 0)),
        out_shape=jax.ShapeDtypeStruct((n, hdim), jnp.float32),
    )(h, agg2, eps11, w1, b1, w2, b2)


def _pool_proj(h, batch2, pw, pb):
    n, hdim = h.shape

    def body(h_ref, b_ref, pw_ref, pb_ref, out_ref):
        seg = b_ref[...]
        ids = lax.broadcasted_iota(jnp.int32, (_G, n), 0)
        onehot = jnp.where(seg == ids, 1.0, 0.0)
        hg = jax.lax.dot(onehot, h_ref[...], preferred_element_type=jnp.float32)
        out_ref[...] = jnp.maximum(hg @ pw_ref[...] + pb_ref[...], 0.0)

    return pl.pallas_call(
        body,
        out_shape=jax.ShapeDtypeStruct((_G, hdim), jnp.float32),
    )(h, batch2, pw, pb)


def kernel(x, edge_attr, params, edge_index, batch):
    n, hdim = x.shape
    E = edge_attr.shape[0]
    npad = ((n + 8 * _NS - 1) // (8 * _NS)) * (8 * _NS)
    epad = _NS * (_TSLOW + _TFAST) * _C - E
    ea_p = jnp.concatenate(
        [edge_attr, jnp.zeros((epad, edge_attr.shape[1]), jnp.float32)])
    e = _edge_mlp(ea_p,
                  params["edge_w1"], params["edge_b1"].reshape(1, -1),
                  params["edge_w2"], params["edge_b2"].reshape(1, -1))
    src_p = jnp.concatenate([edge_index[0], jnp.zeros((epad,), jnp.int32)])
    dst_p = jnp.concatenate([edge_index[1],
                             n + (jnp.arange(epad, dtype=jnp.int32) % (npad - n))])
    idxp = (jnp.left_shift(dst_p, 16) | src_p).reshape(-1, 2 * _C)
    zrows = jnp.zeros((npad // _NS, hdim), jnp.float32)
    h = x
    for lp in params["layers"]:
        agg2 = _msg_agg(h, e, idxp, zrows)
        h = _update(h, agg2, lp["eps"].reshape(1, 1),
                    lp["w1"], lp["b1"].reshape(1, -1),
                    lp["w2"], lp["b2"].reshape(1, -1))
    return _pool_proj(h, batch.reshape(1, -1),
                      params["proj_w"], params["proj_b"].reshape(1, -1))


# speedup vs baseline: 1.4528x; 1.3889x over previous
import jax
import jax.numpy as jnp
import numpy as np
from jax import lax
from jax.experimental import pallas as pl
from jax.experimental.pallas import tpu as pltpu
from jax.experimental.pallas import tpu_sc as plsc

_NC = 2
_NS = 16
_NW = _NC * _NS
_C = 64
_G = 64


def _edge_mlp(ea, w1, b1, w2, b2):
    E, K = ea.shape
    H = w2.shape[1]
    BE = 4096

    def body(ea_ref, w1_ref, b1_ref, w2_ref, b2_ref, out_ref):
        t = jnp.maximum(ea_ref[...] @ w1_ref[...] + b1_ref[...], 0.0)
        out_ref[...] = jnp.maximum(t @ w2_ref[...] + b2_ref[...], 0.0)

    return pl.pallas_call(
        body,
        grid=(E // BE,),
        in_specs=[
            pl.BlockSpec((BE, K), lambda i: (i, 0)),
            pl.BlockSpec(w1.shape, lambda i: (0, 0)),
            pl.BlockSpec(b1.shape, lambda i: (0, 0)),
            pl.BlockSpec(w2.shape, lambda i: (0, 0)),
            pl.BlockSpec(b2.shape, lambda i: (0, 0)),
        ],
        out_specs=pl.BlockSpec((BE, H), lambda i: (i, 0)),
        out_shape=jax.ShapeDtypeStruct((E, H), jnp.float32),
    )(ea, w1, b1, w2, b2)


def _pack16(a):
    r, d = a.shape
    t = a.reshape(r, d // 32, 2, 16).transpose(0, 1, 3, 2)
    return lax.bitcast_convert_type(t, jnp.int32).reshape(r, d // 2)


def _msg_agg(h, e, idxp, zrows):
    n, hdim = h.shape
    T = idxp.shape[1] * 2
    epw = T * _C
    rps = zrows.shape[0]
    npad = rps * _NS
    KU = 4
    mesh = plsc.VectorSubcoreMesh(core_axis_name="c", subcore_axis_name="s")

    def body(h_hbm, e_hbm, idx_hbm, z_hbm, out_hbm,
             idx_v, srcb, dstb, h0, h1, e0, e1,
             agg_sh, hs0, hs1, es0, es1):
        cid = lax.axis_index("c")
        sid = lax.axis_index("s")
        w = cid * _NS + sid

        pltpu.sync_copy(z_hbm, agg_sh.at[pl.ds(sid * rps, rps)])
        pltpu.sync_copy(idx_hbm.at[w], idx_v)
        plsc.subcore_barrier()

        hb = (h0, h1)
        eb = (e0, e1)
        hs = (hs0, hs1)
        es = (es0, es1)

        def unpack(q, b):
            row = q // 2
            for u in range(_C // 16):
                p = idx_v[row, pl.ds(b * _C + u * 16, 16)]
                sl = pl.ds(u * 16, 16)
                srcb[b, sl] = p & 0xFFFF
                dstb[b, sl] = lax.shift_right_logical(p, 16)

        def step(t, b, prefetch):
            pltpu.make_async_copy(
                e_hbm.at[pl.ds(w * epw + t * _C, _C)], eb[b], es[b]).wait()
            pltpu.make_async_copy(h_hbm.at[srcb.at[b]], hb[b], hs[b]).wait()
            hv, ev = hb[b], eb[b]

            def rowblk(i, c2):
                for u in range(KU):
                    r = i * KU + u
                    for cc in range(hdim // 16):
                        sl = pl.ds(cc * 16, 16)
                        hv[r, sl] = jnp.maximum(hv[r, sl] + ev[r, sl], 0.0)
                return c2

            lax.fori_loop(0, _C // KU, rowblk, 0)
            if prefetch:
                pltpu.async_copy(
                    e_hbm.at[pl.ds(w * epw + (t + 2) * _C, _C)], ev, es[b])
            pltpu.sync_copy(hv, agg_sh.at[dstb.at[b]], add=True)
            if prefetch:
                unpack(t + 2, b)
                pltpu.async_copy(h_hbm.at[srcb.at[b]], hv, hs[b])

        for b in (0, 1):
            unpack(b, b)
            pltpu.async_copy(
                e_hbm.at[pl.ds(w * epw + b * _C, _C)], eb[b], es[b])
            pltpu.async_copy(h_hbm.at[srcb.at[b]], hb[b], hs[b])

        def pair(k, c):
            step(2 * k, 0, True)
            step(2 * k + 1, 1, True)
            return c

        lax.fori_loop(0, T // 2 - 1, pair, 0)
        step(T - 2, 0, False)
        step(T - 1, 1, False)

        plsc.subcore_barrier()
        pltpu.sync_copy(agg_sh.at[pl.ds(sid * rps, rps)],
                        out_hbm.at[cid, pl.ds(sid * rps, rps)])

    f = pl.kernel(
        body,
        out_type=jax.ShapeDtypeStruct((_NC, npad, hdim), jnp.float32),
        mesh=mesh,
        scratch_types=[
            pltpu.VMEM((T // 2, 2 * _C), jnp.int32),
            pltpu.VMEM((2, _C), jnp.int32),
            pltpu.VMEM((2, _C), jnp.int32),
            pltpu.VMEM((_C, hdim), jnp.float32),
            pltpu.VMEM((_C, hdim), jnp.float32),
            pltpu.VMEM((_C, hdim), jnp.float32),
            pltpu.VMEM((_C, hdim), jnp.float32),
            pltpu.VMEM_SHARED((npad, hdim), jnp.float32),
            pltpu.SemaphoreType.DMA,
            pltpu.SemaphoreType.DMA,
            pltpu.SemaphoreType.DMA,
            pltpu.SemaphoreType.DMA,
        ],
    )
    return f(h, e, idxp, zrows)


def _update(h, agg2, eps11, w1, b1, w2, b2):
    n, hdim = h.shape
    BN = 2000

    def body(h_ref, a_ref, eps_ref, w1_ref, b1_ref, w2_ref, b2_ref, out_ref):
        scale = 1.0 + eps_ref[0, 0]
        z = scale * h_ref[...] + a_ref[0] + a_ref[1]
        z = jnp.maximum(z @ w1_ref[...] + b1_ref[...], 0.0)
        out_ref[...] = jnp.maximum(z @ w2_ref[...] + b2_ref[...], 0.0)

    return pl.pallas_call(
        body,
        grid=(n // BN,),
        in_specs=[
            pl.BlockSpec((BN, hdim), lambda i: (i, 0)),
            pl.BlockSpec((2, BN, hdim), lambda i: (0, i, 0)),
            pl.BlockSpec((1, 1), lambda i: (0, 0), memory_space=pltpu.SMEM),
            pl.BlockSpec((hdim, hdim), lambda i: (0, 0)),
            pl.BlockSpec((1, hdim), lambda i: (0, 0)),
            pl.BlockSpec((hdim, hdim), lambda i: (0, 0)),
            pl.BlockSpec((1, hdim), lambda i: (0, 0)),
        ],
        out_specs=pl.BlockSpec((BN, hdim), lambda i: (i, 0)),
        out_shape=jax.ShapeDtypeStruct((n, hdim), jnp.float32),
    )(h, agg2, eps11, w1, b1, w2, b2)


def _pool_proj(h, batch2, pw, pb):
    n, hdim = h.shape

    def body(h_ref, b_ref, pw_ref, pb_ref, out_ref):
        seg = b_ref[...]
        ids = lax.broadcasted_iota(jnp.int32, (_G, n), 0)
        onehot = jnp.where(seg == ids, 1.0, 0.0)
        hg = jax.lax.dot(onehot, h_ref[...], preferred_element_type=jnp.float32)
        out_ref[...] = jnp.maximum(hg @ pw_ref[...] + pb_ref[...], 0.0)

    return pl.pallas_call(
        body,
        out_shape=jax.ShapeDtypeStruct((_G, hdim), jnp.float32),
    )(h, batch2, pw, pb)


def kernel(x, edge_attr, params, edge_index, batch):
    n, hdim = x.shape
    E = edge_attr.shape[0]
    npad = ((n + 8 * _NS - 1) // (8 * _NS)) * (8 * _NS)
    T = -(-E // (2 * _NW * _C)) * 2
    epad = _NW * T * _C - E
    ea_p = jnp.concatenate(
        [edge_attr, jnp.zeros((epad, edge_attr.shape[1]), jnp.float32)])
    e = _edge_mlp(ea_p,
                  params["edge_w1"], params["edge_b1"].reshape(1, -1),
                  params["edge_w2"], params["edge_b2"].reshape(1, -1))
    src_p = jnp.concatenate([edge_index[0], jnp.zeros((epad,), jnp.int32)])
    dst_p = jnp.concatenate([edge_index[1],
                             n + (jnp.arange(epad, dtype=jnp.int32) % (npad - n))])
    idxp = (jnp.left_shift(dst_p, 16) | src_p).reshape(_NW, T // 2, 2 * _C)
    zrows = jnp.zeros((npad // _NS, hdim), jnp.float32)
    h = x
    for lp in params["layers"]:
        agg2 = _msg_agg(h, e, idxp, zrows)
        h = _update(h, agg2, lp["eps"].reshape(1, 1),
                    lp["w1"], lp["b1"].reshape(1, -1),
                    lp["w2"], lp["b2"].reshape(1, -1))
    return _pool_proj(h, batch.reshape(1, -1),
                      params["proj_w"], params["proj_b"].reshape(1, -1))


# speedup vs baseline: 1.4568x; 1.0028x over previous
import jax
import jax.numpy as jnp
from jax import lax
from jax.experimental import pallas as pl
from jax.experimental.pallas import tpu as pltpu
from jax.experimental.pallas import tpu_sc as plsc

_NC = 2
_NS = 16
_NW = _NC * _NS
_C = 64
_G = 64


def _edge_mlp(ea, w1, b1, w2, b2):
    E, K = ea.shape
    H = w2.shape[1]
    BE = 4096

    def body(ea_ref, w1_ref, b1_ref, w2_ref, b2_ref, out_ref):
        t = jnp.maximum(ea_ref[...] @ w1_ref[...] + b1_ref[...], 0.0)
        out_ref[...] = jnp.maximum(t @ w2_ref[...] + b2_ref[...], 0.0)

    return pl.pallas_call(
        body,
        grid=(E // BE,),
        in_specs=[
            pl.BlockSpec((BE, K), lambda i: (i, 0)),
            pl.BlockSpec(w1.shape, lambda i: (0, 0)),
            pl.BlockSpec(b1.shape, lambda i: (0, 0)),
            pl.BlockSpec(w2.shape, lambda i: (0, 0)),
            pl.BlockSpec(b2.shape, lambda i: (0, 0)),
        ],
        out_specs=pl.BlockSpec((BE, H), lambda i: (i, 0)),
        out_shape=jax.ShapeDtypeStruct((E, H), jnp.float32),
    )(ea, w1, b1, w2, b2)


def _msg_agg(h, e, idxp, zrows):
    n, hdim = h.shape
    T = idxp.shape[1] * 2
    epw = T * _C
    rps = zrows.shape[0]
    npad = rps * _NS
    KU = 4
    mesh = plsc.VectorSubcoreMesh(core_axis_name="c", subcore_axis_name="s")

    def body(h_hbm, e_hbm, idx_hbm, z_hbm, out_hbm,
             idx_v, srcb, dstb, h0, h1, e0, e1,
             agg_sh, hs0, hs1, es0, es1):
        cid = lax.axis_index("c")
        sid = lax.axis_index("s")
        w = cid * _NS + sid

        pltpu.sync_copy(z_hbm, agg_sh.at[pl.ds(sid * rps, rps)])
        pltpu.sync_copy(idx_hbm.at[w], idx_v)
        plsc.subcore_barrier()

        hb = (h0, h1)
        eb = (e0, e1)
        hs = (hs0, hs1)
        es = (es0, es1)

        def unpack(q, b):
            row = q // 2
            for u in range(_C // 16):
                p = idx_v[row, pl.ds(b * _C + u * 16, 16)]
                sl = pl.ds(u * 16, 16)
                srcb[b, sl] = p & 0xFFFF
                dstb[b, sl] = lax.shift_right_logical(p, 16)

        def step(t, b, prefetch):
            pltpu.make_async_copy(
                e_hbm.at[pl.ds(w * epw + t * _C, _C)], eb[b], es[b]).wait()
            pltpu.make_async_copy(h_hbm.at[srcb.at[b]], hb[b], hs[b]).wait()
            hv, ev = hb[b], eb[b]

            def rowblk(i, c2):
                for u in range(KU):
                    r = i * KU + u
                    for cc in range(hdim // 16):
                        sl = pl.ds(cc * 16, 16)
                        hv[r, sl] = jnp.maximum(hv[r, sl] + ev[r, sl], 0.0)
                return c2

            lax.fori_loop(0, _C // KU, rowblk, 0)
            if prefetch:
                pltpu.async_copy(
                    e_hbm.at[pl.ds(w * epw + (t + 2) * _C, _C)], ev, es[b])
            pltpu.sync_copy(hv, agg_sh.at[dstb.at[b]], add=True)
            if prefetch:
                unpack(t + 2, b)
                pltpu.async_copy(h_hbm.at[srcb.at[b]], hv, hs[b])

        for b in (0, 1):
            unpack(b, b)
            pltpu.async_copy(
                e_hbm.at[pl.ds(w * epw + b * _C, _C)], eb[b], es[b])
            pltpu.async_copy(h_hbm.at[srcb.at[b]], hb[b], hs[b])

        def pair(k, c):
            step(2 * k, 0, True)
            step(2 * k + 1, 1, True)
            return c

        lax.fori_loop(0, T // 2 - 1, pair, 0)
        step(T - 2, 0, False)
        step(T - 1, 1, False)

        plsc.subcore_barrier()
        pltpu.sync_copy(agg_sh.at[pl.ds(sid * rps, rps)],
                        out_hbm.at[cid, pl.ds(sid * rps, rps)])

    f = pl.kernel(
        body,
        out_type=jax.ShapeDtypeStruct((_NC, npad, hdim), jnp.float32),
        mesh=mesh,
        scratch_types=[
            pltpu.VMEM((T // 2, 2 * _C), jnp.int32),
            pltpu.VMEM((2, _C), jnp.int32),
            pltpu.VMEM((2, _C), jnp.int32),
            pltpu.VMEM((_C, hdim), jnp.float32),
            pltpu.VMEM((_C, hdim), jnp.float32),
            pltpu.VMEM((_C, hdim), jnp.float32),
            pltpu.VMEM((_C, hdim), jnp.float32),
            pltpu.VMEM_SHARED((npad, hdim), jnp.float32),
            pltpu.SemaphoreType.DMA,
            pltpu.SemaphoreType.DMA,
            pltpu.SemaphoreType.DMA,
            pltpu.SemaphoreType.DMA,
        ],
    )
    return f(h, e, idxp, zrows)


def _update(h, agg2, eps11, w1, b1, w2, b2):
    n, hdim = h.shape
    BN = 2000

    def body(h_ref, a_ref, eps_ref, w1_ref, b1_ref, w2_ref, b2_ref, out_ref):
        scale = 1.0 + eps_ref[0, 0]
        z = scale * h_ref[...] + a_ref[0] + a_ref[1]
        z = jnp.maximum(z @ w1_ref[...] + b1_ref[...], 0.0)
        out_ref[...] = jnp.maximum(z @ w2_ref[...] + b2_ref[...], 0.0)

    return pl.pallas_call(
        body,
        grid=(n // BN,),
        in_specs=[
            pl.BlockSpec((BN, hdim), lambda i: (i, 0)),
            pl.BlockSpec((2, BN, hdim), lambda i: (0, i, 0)),
            pl.BlockSpec((1, 1), lambda i: (0, 0), memory_space=pltpu.SMEM),
            pl.BlockSpec((hdim, hdim), lambda i: (0, 0)),
            pl.BlockSpec((1, hdim), lambda i: (0, 0)),
            pl.BlockSpec((hdim, hdim), lambda i: (0, 0)),
            pl.BlockSpec((1, hdim), lambda i: (0, 0)),
        ],
        out_specs=pl.BlockSpec((BN, hdim), lambda i: (i, 0)),
        out_shape=jax.ShapeDtypeStruct((n, hdim), jnp.float32),
    )(h, agg2, eps11, w1, b1, w2, b2)


def _pool_proj(h, batch2, pw, pb):
    n, hdim = h.shape

    def body(h_ref, b_ref, pw_ref, pb_ref, out_ref):
        seg = b_ref[...]
        ids = lax.broadcasted_iota(jnp.int32, (_G, n), 0)
        onehot = jnp.where(seg == ids, 1.0, 0.0)
        hg = jax.lax.dot(onehot, h_ref[...], preferred_element_type=jnp.float32)
        out_ref[...] = jnp.maximum(hg @ pw_ref[...] + pb_ref[...], 0.0)

    return pl.pallas_call(
        body,
        out_shape=jax.ShapeDtypeStruct((_G, hdim), jnp.float32),
    )(h, batch2, pw, pb)


def kernel(x, edge_attr, params, edge_index, batch):
    n, hdim = x.shape
    E = edge_attr.shape[0]
    npad = ((n + 8 * _NS - 1) // (8 * _NS)) * (8 * _NS)
    T = -(-E // (2 * _NW * _C)) * 2
    epad = _NW * T * _C - E
    ea_p = jnp.concatenate(
        [edge_attr, jnp.zeros((epad, edge_attr.shape[1]), jnp.float32)])
    e = _edge_mlp(ea_p,
                  params["edge_w1"], params["edge_b1"].reshape(1, -1),
                  params["edge_w2"], params["edge_b2"].reshape(1, -1))
    src_p = jnp.concatenate([edge_index[0], jnp.zeros((epad,), jnp.int32)])
    dst_p = jnp.concatenate([edge_index[1],
                             n + (jnp.arange(epad, dtype=jnp.int32) % (npad - n))])
    idxp = (jnp.left_shift(dst_p, 16) | src_p).reshape(_NW, T // 2, 2 * _C)
    zrows = jnp.zeros((npad // _NS, hdim), jnp.float32)
    h = x
    for lp in params["layers"]:
        agg2 = _msg_agg(h, e, idxp, zrows)
        h = _update(h, agg2, lp["eps"].reshape(1, 1),
                    lp["w1"], lp["b1"].reshape(1, -1),
                    lp["w2"], lp["b2"].reshape(1, -1))
    return _pool_proj(h, batch.reshape(1, -1),
                      params["proj_w"], params["proj_b"].reshape(1, -1))
